# baseline (speedup 1.0000x reference)
import functools

import jax
import jax.numpy as jnp
from jax import lax
from jax.experimental import pallas as pl
from jax.experimental.pallas import tpu as pltpu
from jax.experimental.pallas import tpu_sc as plsc

E = 8
K = 2
D = 1024
F = 2048
N = 2048
EP = 128
C = N
BR = 128
NB = C // BR
NBLK = 40

NW = 32
PAIRS = K * N
PPW = PAIRS // NW
TPW = N // NW


def _router_kernel(x_ref, rw_ref, misc_ref, bd_ref, loss_ref):
    x = x_ref[...]
    rw = rw_ref[...]
    logits = jnp.dot(x, rw, preferred_element_type=jnp.float32)
    col = lax.broadcasted_iota(jnp.int32, (N, EP), 1)
    valid = col < E
    logits = jnp.where(valid, logits, -1e30)
    m = jnp.max(logits, axis=1, keepdims=True)
    ex = jnp.where(valid, jnp.exp(logits - m), 0.0)
    s = jnp.sum(ex, axis=1, keepdims=True)
    probs = ex / s
    p1 = jnp.max(probs, axis=1, keepdims=True)
    i1 = jnp.min(jnp.where(probs == p1, col, EP), axis=1, keepdims=True)
    oh1 = (col == i1).astype(jnp.float32)
    probs2 = jnp.where(col == i1, -1.0, probs)
    p2 = jnp.max(probs2, axis=1, keepdims=True)
    i2 = jnp.min(jnp.where(probs2 == p2, col, EP), axis=1, keepdims=True)
    oh2 = (col == i2).astype(jnp.float32)
    tot = p1 + p2
    wv1 = p1 / tot
    wv2 = p2 / tot
    a = oh1 + oh2
    incl = a
    sh = 1
    while sh < N:
        incl = incl + jnp.concatenate(
            [jnp.zeros((sh, EP), jnp.float32), incl[:N - sh, :]], axis=0)
        sh *= 2
    excl = incl - a
    slot1 = jnp.sum(jnp.where(col == i1, excl, 0.0), axis=1, keepdims=True)
    slot2 = jnp.sum(jnp.where(col == i2, excl, 0.0), axis=1, keepdims=True)
    dst1 = i1.astype(jnp.float32) * C + slot1
    dst2 = i2.astype(jnp.float32) * C + slot2
    misc = (jnp.where(col == 0, dst1, 0.0) + jnp.where(col == 1, dst2, 0.0)
            + jnp.where((col >= 16) & (col < 32), wv1, 0.0)
            + jnp.where((col >= 32) & (col < 48), wv2, 0.0))
    misc_ref[...] = misc
    me = jnp.sum(probs, axis=0, keepdims=True) / N
    cntf = jnp.sum(a, axis=0, keepdims=True)
    ce = cntf / (N * K)
    loss_ref[0, 0] = E * jnp.sum(me * ce)
    nb = jnp.floor((cntf + (BR - 1)) / BR)
    lt = (lax.broadcasted_iota(jnp.int32, (EP, EP), 0) <
          lax.broadcasted_iota(jnp.int32, (EP, EP), 1)).astype(jnp.float32)
    off = jnp.dot(nb, lt, preferred_element_type=jnp.float32)
    total = jnp.sum(nb)
    bi_i = lax.broadcasted_iota(jnp.int32, (64, EP), 0).astype(jnp.float32)
    e_i = lax.broadcasted_iota(jnp.int32, (64, EP), 1)
    cmp = (off <= bi_i) & (e_i < E)
    be = jnp.sum(cmp.astype(jnp.float32), axis=1, keepdims=True) - 1.0
    offsel = jnp.max(jnp.where(cmp, off * jnp.ones((64, EP), jnp.float32),
                               0.0), axis=1, keepdims=True)
    iv = bi_i[:, :1]
    bv = be * NB + (iv - offsel)
    blast = jnp.sum(jnp.where(iv == total - 1.0, bv, 0.0),
                    axis=0, keepdims=True)
    bv = jnp.where(iv < total, bv, blast)
    r64 = lax.broadcasted_iota(jnp.int32, (64, EP), 0)
    c64 = lax.broadcasted_iota(jnp.int32, (64, EP), 1)
    bd_row = jnp.sum(jnp.where(r64 == c64, bv, 0.0), axis=0, keepdims=True)
    lane = lax.broadcasted_iota(jnp.int32, (1, EP), 1)
    bd_ref[...] = bd_row + jnp.where(lane == 64, total, 0.0)


def _run_router(x, router_w):
    rw_pad = jnp.pad(router_w, ((0, 0), (0, EP - E)))
    return pl.pallas_call(
        _router_kernel,
        out_shape=[
            jax.ShapeDtypeStruct((N, EP), jnp.float32),
            jax.ShapeDtypeStruct((1, EP), jnp.float32),
            jax.ShapeDtypeStruct((1, 1), jnp.float32),
        ],
        out_specs=[
            pl.BlockSpec((N, EP), lambda: (0, 0)),
            pl.BlockSpec((1, EP), lambda: (0, 0)),
            pl.BlockSpec(memory_space=pltpu.SMEM),
        ],
    )(x, rw_pad)


_DCH = 64
_DNCH = PPW // _DCH


@functools.cache
def _make_dispatch_kernel():
    mesh = plsc.VectorSubcoreMesh(core_axis_name="c", subcore_axis_name="s")

    @functools.partial(
        pl.kernel,
        out_type=jax.ShapeDtypeStruct((E * C, D), jnp.float32),
        mesh=mesh,
        scratch_types=[
            pltpu.VMEM((_DCH,), jnp.int32),
            pltpu.VMEM((_DCH, D), jnp.float32),
            pltpu.SemaphoreType.DMA,
        ],
    )
    def _dispatch_kernel(x_hbm, dst_hbm, xbuf_hbm, idx_v, rows_v, sem):
        wid = lax.axis_index("s") * 2 + lax.axis_index("c")
        p0 = wid * PPW
        t0 = jnp.where(wid < NW // 2, p0, p0 - N)
        for ch in range(_DNCH):
            pltpu.sync_copy(dst_hbm.at[pl.ds(p0 + ch * _DCH, _DCH)], idx_v)
            pltpu.sync_copy(x_hbm.at[pl.ds(t0 + ch * _DCH, _DCH)], rows_v)
            pltpu.async_copy(rows_v, xbuf_hbm.at[idx_v], sem).wait()

    return _dispatch_kernel


def _ffn_kernel(bi_ref, tot_ref, x_ref, w1_ref, b1_ref, w2_ref, b2_ref,
                y_ref):
    i = pl.program_id(0)

    @pl.when(i < tot_ref[0])
    def _active():
        h = jnp.dot(x_ref[...].astype(jnp.bfloat16), w1_ref[0],
                    preferred_element_type=jnp.float32)
        h = jnp.maximum(h + b1_ref[0], 0.0).astype(jnp.bfloat16)
        y = jnp.dot(h, w2_ref[0], preferred_element_type=jnp.float32)
        y_ref[...] = y + b2_ref[0]


def _run_ffn(bi_arr, tot_arr, x_buf, w1, b1, w2, b2):
    def xb_map(i, bi, tot):
        return (bi[i], 0)

    def w_map(i, bi, tot):
        return (bi[i] // NB, 0, 0)

    grid_spec = pltpu.PrefetchScalarGridSpec(
        num_scalar_prefetch=2,
        grid=(NBLK,),
        in_specs=[
            pl.BlockSpec((BR, D), xb_map),
            pl.BlockSpec((1, D, F), w_map),
            pl.BlockSpec((1, 1, F), w_map),
            pl.BlockSpec((1, F, D), w_map),
            pl.BlockSpec((1, 1, D), w_map),
        ],
        out_specs=pl.BlockSpec((BR, D), xb_map),
    )
    return pl.pallas_call(
        _ffn_kernel,
        grid_spec=grid_spec,
        out_shape=jax.ShapeDtypeStruct((E * C, D), jnp.float32),
    )(bi_arr, tot_arr, x_buf, w1, b1.reshape(E, 1, F), w2,
      b2.reshape(E, 1, D))


_CCH = 32
_CNCH = TPW // _CCH


@functools.cache
def _make_combine_kernel():
    mesh = plsc.VectorSubcoreMesh(core_axis_name="c", subcore_axis_name="s")

    @functools.partial(
        pl.kernel,
        out_type=jax.ShapeDtypeStruct((N, D), jnp.float32),
        mesh=mesh,
        scratch_types=[
            pltpu.VMEM((_CCH,), jnp.int32),
            pltpu.VMEM((_CCH,), jnp.int32),
            pltpu.VMEM((_CCH, EP), jnp.float32),
            pltpu.VMEM((_CCH, D), jnp.float32),
            pltpu.VMEM((_CCH, D), jnp.float32),
            pltpu.SemaphoreType.DMA,
        ],
    )
    def _combine_kernel(ybuf_hbm, d1_hbm, d2_hbm, misc_hbm, out_hbm,
                        i1_v, i2_v, wv_v, g1_v, g2_v, sem):
        wid = lax.axis_index("s") * 2 + lax.axis_index("c")
        base = wid * TPW
        for ch in range(_CNCH):
            b = base + ch * _CCH
            pltpu.sync_copy(d1_hbm.at[pl.ds(b, _CCH)], i1_v)
            pltpu.sync_copy(d2_hbm.at[pl.ds(b, _CCH)], i2_v)
            pltpu.sync_copy(misc_hbm.at[pl.ds(b, _CCH)], wv_v)
            pltpu.async_copy(ybuf_hbm.at[i1_v], g1_v, sem).wait()
            pltpu.async_copy(ybuf_hbm.at[i2_v], g2_v, sem).wait()

            def row_body(i, _):
                a1 = wv_v[i, pl.ds(16, 16)]
                a2 = wv_v[i, pl.ds(32, 16)]
                for cidx in range(D // 16):
                    sl = pl.ds(cidx * 16, 16)
                    g1_v[i, sl] = g1_v[i, sl] * a1 + g2_v[i, sl] * a2
                return 0

            lax.fori_loop(0, _CCH, row_body, 0)
            pltpu.sync_copy(g1_v, out_hbm.at[pl.ds(b, _CCH)])

    return _combine_kernel


def kernel(input_batch, router_w, w1, b1, w2, b2):
    misc, bd, loss = _run_router(input_batch, router_w)
    dst = jnp.concatenate([misc[:, 0], misc[:, 1]]).astype(jnp.int32)
    x_buf = _make_dispatch_kernel()(input_batch, dst)
    bi_arr = bd[0, :NBLK].astype(jnp.int32)
    tot_arr = bd[0, 64:65].astype(jnp.int32)
    y_buf = _run_ffn(bi_arr, tot_arr, x_buf,
                     w1.astype(jnp.bfloat16), b1, w2.astype(jnp.bfloat16), b2)
    out = _make_combine_kernel()(y_buf,
                                 misc[:, 0].astype(jnp.int32),
                                 misc[:, 1].astype(jnp.int32),
                                 misc)
    return out, loss[0, 0]


# speedup vs baseline: 1.2532x; 1.2532x over previous
import functools

import jax
import jax.numpy as jnp
from jax import lax
from jax.experimental import pallas as pl
from jax.experimental.pallas import tpu as pltpu
from jax.experimental.pallas import tpu_sc as plsc

E = 8
K = 2
D = 1024
F = 2048
N = 2048
EP = 128
C = N
BR = 128
NB = C // BR
NBLK = 40

NW = 32
PAIRS = K * N
PPW = PAIRS // NW
TPW = N // NW


def _router_kernel(x_ref, rw_ref, misc_ref, bd_ref, loss_ref):
    x = x_ref[...]
    rw = rw_ref[...]
    logits = jnp.dot(x, rw, preferred_element_type=jnp.float32)
    col = lax.broadcasted_iota(jnp.int32, (N, EP), 1)
    valid = col < E
    logits = jnp.where(valid, logits, -1e30)
    m = jnp.max(logits, axis=1, keepdims=True)
    ex = jnp.where(valid, jnp.exp(logits - m), 0.0)
    s = jnp.sum(ex, axis=1, keepdims=True)
    probs = ex / s
    p1 = jnp.max(probs, axis=1, keepdims=True)
    i1 = jnp.min(jnp.where(probs == p1, col, EP), axis=1, keepdims=True)
    oh1 = (col == i1).astype(jnp.float32)
    probs2 = jnp.where(col == i1, -1.0, probs)
    p2 = jnp.max(probs2, axis=1, keepdims=True)
    i2 = jnp.min(jnp.where(probs2 == p2, col, EP), axis=1, keepdims=True)
    oh2 = (col == i2).astype(jnp.float32)
    tot = p1 + p2
    wv1 = p1 / tot
    wv2 = p2 / tot
    a = oh1 + oh2
    incl = a
    sh = 1
    while sh < N:
        incl = incl + jnp.concatenate(
            [jnp.zeros((sh, EP), jnp.float32), incl[:N - sh, :]], axis=0)
        sh *= 2
    excl = incl - a
    slot1 = jnp.sum(jnp.where(col == i1, excl, 0.0), axis=1, keepdims=True)
    slot2 = jnp.sum(jnp.where(col == i2, excl, 0.0), axis=1, keepdims=True)
    dst1 = i1.astype(jnp.float32) * C + slot1
    dst2 = i2.astype(jnp.float32) * C + slot2
    misc = (jnp.where(col == 0, dst1, 0.0) + jnp.where(col == 1, dst2, 0.0)
            + jnp.where((col >= 16) & (col < 32), wv1, 0.0)
            + jnp.where((col >= 32) & (col < 48), wv2, 0.0))
    misc_ref[...] = misc
    me = jnp.sum(probs, axis=0, keepdims=True) / N
    cntf = jnp.sum(a, axis=0, keepdims=True)
    ce = cntf / (N * K)
    loss_ref[0, 0] = E * jnp.sum(me * ce)
    nb = jnp.floor((cntf + (BR - 1)) / BR)
    lt = (lax.broadcasted_iota(jnp.int32, (EP, EP), 0) <
          lax.broadcasted_iota(jnp.int32, (EP, EP), 1)).astype(jnp.float32)
    off = jnp.dot(nb, lt, preferred_element_type=jnp.float32)
    total = jnp.sum(nb)
    bi_i = lax.broadcasted_iota(jnp.int32, (64, EP), 0).astype(jnp.float32)
    e_i = lax.broadcasted_iota(jnp.int32, (64, EP), 1)
    cmp = (off <= bi_i) & (e_i < E)
    be = jnp.sum(cmp.astype(jnp.float32), axis=1, keepdims=True) - 1.0
    offsel = jnp.max(jnp.where(cmp, off * jnp.ones((64, EP), jnp.float32),
                               0.0), axis=1, keepdims=True)
    iv = bi_i[:, :1]
    bv = be * NB + (iv - offsel)
    blast = jnp.sum(jnp.where(iv == total - 1.0, bv, 0.0),
                    axis=0, keepdims=True)
    bv = jnp.where(iv < total, bv, blast)
    r64 = lax.broadcasted_iota(jnp.int32, (64, EP), 0)
    c64 = lax.broadcasted_iota(jnp.int32, (64, EP), 1)
    bd_row = jnp.sum(jnp.where(r64 == c64, bv, 0.0), axis=0, keepdims=True)
    lane = lax.broadcasted_iota(jnp.int32, (1, EP), 1)
    bd_ref[...] = bd_row + jnp.where(lane == 64, total, 0.0)


def _run_router(x, router_w):
    rw_pad = jnp.pad(router_w, ((0, 0), (0, EP - E)))
    return pl.pallas_call(
        _router_kernel,
        out_shape=[
            jax.ShapeDtypeStruct((N, EP), jnp.float32),
            jax.ShapeDtypeStruct((1, EP), jnp.float32),
            jax.ShapeDtypeStruct((1, 1), jnp.float32),
        ],
        out_specs=[
            pl.BlockSpec((N, EP), lambda: (0, 0)),
            pl.BlockSpec((1, EP), lambda: (0, 0)),
            pl.BlockSpec(memory_space=pltpu.SMEM),
        ],
    )(x, rw_pad)


_DCH = 32
_DNCH = PPW // _DCH


@functools.cache
def _make_dispatch_kernel():
    mesh = plsc.VectorSubcoreMesh(core_axis_name="c", subcore_axis_name="s")

    @functools.partial(
        pl.kernel,
        out_type=jax.ShapeDtypeStruct((E * C, D), jnp.float32),
        mesh=mesh,
        scratch_types=[
            pltpu.VMEM((_DCH,), jnp.int32),
            pltpu.VMEM((_DCH,), jnp.int32),
            pltpu.VMEM((_DCH, D), jnp.float32),
            pltpu.VMEM((_DCH, D), jnp.float32),
            pltpu.SemaphoreType.DMA,
            pltpu.SemaphoreType.DMA,
            pltpu.SemaphoreType.DMA,
            pltpu.SemaphoreType.DMA,
        ],
    )
    def _dispatch_kernel(x_hbm, dst_hbm, xbuf_hbm,
                         idx_a, idx_b, rows_a, rows_b,
                         sem_la, sem_lb, sem_sa, sem_sb):
        wid = lax.axis_index("s") * 2 + lax.axis_index("c")
        p0 = wid * PPW
        t0 = jnp.where(wid < NW // 2, p0, p0 - N)
        idx = (idx_a, idx_b)
        rows = (rows_a, rows_b)
        sem_l = (sem_la, sem_lb)
        sem_s = (sem_sa, sem_sb)

        def load(ch):
            s = ch % 2
            pltpu.sync_copy(dst_hbm.at[pl.ds(p0 + ch * _DCH, _DCH)], idx[s])
            return pltpu.async_copy(
                x_hbm.at[pl.ds(t0 + ch * _DCH, _DCH)], rows[s], sem_l[s])

        pend = load(0)
        scat = [None, None]
        for ch in range(_DNCH):
            s = ch % 2
            nxt = None
            if ch + 1 < _DNCH:
                if scat[(ch + 1) % 2] is not None:
                    scat[(ch + 1) % 2].wait()
                    scat[(ch + 1) % 2] = None
                nxt = load(ch + 1)
            pend.wait()
            scat[s] = pltpu.async_copy(rows[s], xbuf_hbm.at[idx[s]], sem_s[s])
            pend = nxt
        for s in range(2):
            if scat[s] is not None:
                scat[s].wait()

    return _dispatch_kernel


def _ffn_kernel(bi_ref, tot_ref, x_ref, w1_ref, b1_ref, w2_ref, b2_ref,
                y_ref):
    i = pl.program_id(0)

    @pl.when(i < tot_ref[0])
    def _active():
        h = jnp.dot(x_ref[...], w1_ref[0], preferred_element_type=jnp.float32)
        h = jnp.maximum(h + b1_ref[0], 0.0)
        y = jnp.dot(h, w2_ref[0], preferred_element_type=jnp.float32)
        y_ref[...] = y + b2_ref[0]


def _run_ffn(bi_arr, tot_arr, x_buf, w1, b1, w2, b2):
    def xb_map(i, bi, tot):
        return (bi[i], 0)

    def w_map(i, bi, tot):
        return (bi[i] // NB, 0, 0)

    grid_spec = pltpu.PrefetchScalarGridSpec(
        num_scalar_prefetch=2,
        grid=(NBLK,),
        in_specs=[
            pl.BlockSpec((BR, D), xb_map),
            pl.BlockSpec((1, D, F), w_map),
            pl.BlockSpec((1, 1, F), w_map),
            pl.BlockSpec((1, F, D), w_map),
            pl.BlockSpec((1, 1, D), w_map),
        ],
        out_specs=pl.BlockSpec((BR, D), xb_map),
    )
    return pl.pallas_call(
        _ffn_kernel,
        grid_spec=grid_spec,
        out_shape=jax.ShapeDtypeStruct((E * C, D), jnp.float32),
    )(bi_arr, tot_arr, x_buf, w1, b1.reshape(E, 1, F), w2,
      b2.reshape(E, 1, D))


_CCH = 16
_CNCH = TPW // _CCH


@functools.cache
def _make_combine_kernel():
    mesh = plsc.VectorSubcoreMesh(core_axis_name="c", subcore_axis_name="s")

    @functools.partial(
        pl.kernel,
        out_type=jax.ShapeDtypeStruct((N, D), jnp.float32),
        mesh=mesh,
        scratch_types=(
            [pltpu.VMEM((_CCH,), jnp.int32)] * 4
            + [pltpu.VMEM((_CCH, EP), jnp.float32)] * 2
            + [pltpu.VMEM((_CCH, D), jnp.float32)] * 4
            + [pltpu.SemaphoreType.DMA] * 2
        ),
    )
    def _combine_kernel(ybuf_hbm, d1_hbm, d2_hbm, misc_hbm, out_hbm,
                        i1_a, i1_b, i2_a, i2_b, wv_a, wv_b,
                        g1_a, g1_b, g2_a, g2_b, sem_a, sem_b):
        wid = lax.axis_index("s") * 2 + lax.axis_index("c")
        base = wid * TPW
        i1 = (i1_a, i1_b)
        i2 = (i2_a, i2_b)
        wv = (wv_a, wv_b)
        g1 = (g1_a, g1_b)
        g2 = (g2_a, g2_b)
        sems = (sem_a, sem_b)

        def load(ch):
            s = ch % 2
            b = base + ch * _CCH
            pltpu.sync_copy(d1_hbm.at[pl.ds(b, _CCH)], i1[s])
            pltpu.sync_copy(d2_hbm.at[pl.ds(b, _CCH)], i2[s])
            pltpu.sync_copy(misc_hbm.at[pl.ds(b, _CCH)], wv[s])
            c1 = pltpu.async_copy(ybuf_hbm.at[i1[s]], g1[s], sems[s])
            c2 = pltpu.async_copy(ybuf_hbm.at[i2[s]], g2[s], sems[s])
            return c1, c2

        pend = load(0)
        for ch in range(_CNCH):
            s = ch % 2
            nxt = load(ch + 1) if ch + 1 < _CNCH else None
            pend[0].wait()
            pend[1].wait()
            g1s, g2s, wvs = g1[s], g2[s], wv[s]

            def row_body(i, _):
                a1 = wvs[i, pl.ds(16, 16)]
                a2 = wvs[i, pl.ds(32, 16)]
                for cidx in range(D // 16):
                    sl = pl.ds(cidx * 16, 16)
                    g1s[i, sl] = g1s[i, sl] * a1 + g2s[i, sl] * a2
                return 0

            lax.fori_loop(0, _CCH, row_body, 0)
            pltpu.sync_copy(g1s, out_hbm.at[pl.ds(base + ch * _CCH, _CCH)])
            pend = nxt

    return _combine_kernel


def kernel(input_batch, router_w, w1, b1, w2, b2):
    misc, bd, loss = _run_router(input_batch, router_w)
    dst = jnp.concatenate([misc[:, 0], misc[:, 1]]).astype(jnp.int32)
    x_buf = _make_dispatch_kernel()(input_batch, dst)
    bi_arr = bd[0, :NBLK].astype(jnp.int32)
    tot_arr = bd[0, 64:65].astype(jnp.int32)
    y_buf = _run_ffn(bi_arr, tot_arr, x_buf, w1, b1, w2, b2)
    out = _make_combine_kernel()(y_buf,
                                 misc[:, 0].astype(jnp.int32),
                                 misc[:, 1].astype(jnp.int32),
                                 misc)
    return out, loss[0, 0]
